# baseline (speedup 1.0000x reference)
import functools

import jax
import jax.numpy as jnp
from jax import lax
from jax.experimental import pallas as pl
from jax.experimental.pallas import tpu as pltpu
from jax.experimental.pallas import tpu_sc as plsc

D = 64
B_ROWS = 4096
SEQ = 200
NC = 2
NS = 16
NW = NC * NS
RPW = B_ROWS // NW
CPW = RPW * 2
NB = 8
LA = 4

_mesh = plsc.VectorSubcoreMesh(core_axis_name="c", subcore_axis_name="s")


def _c_size(b):
    return 128 if b % 2 == 0 else SEQ - 128


def _c_off(b):
    return 0 if b % 2 == 0 else 128


@functools.partial(
    pl.kernel,
    out_type=jax.ShapeDtypeStruct((B_ROWS, SEQ, D), jnp.float32),
    mesh=_mesh,
    scratch_types=[
        pltpu.VMEM((RPW, SEQ), jnp.int32),
        [pltpu.VMEM((_c_size(b), D), jnp.float32) for b in range(NB)],
        [pltpu.SemaphoreType.DMA] * NB,
        [pltpu.SemaphoreType.DMA] * NB,
    ],
    compiler_params=pltpu.CompilerParams(use_tc_tiling_on_sc=False),
)
def _gather_kernel(idx_hbm, table_hbm, out_hbm, idx_v, rows, sem_g, sem_o):
    wid = lax.axis_index("s") * NC + lax.axis_index("c")
    row0 = wid * RPW
    pltpu.sync_copy(idx_hbm.at[pl.ds(row0, RPW)], idx_v)

    def idx_ref(k, b):
        return idx_v.at[k // 2, pl.ds(_c_off(b), _c_size(b))]

    def out_ref(k, b):
        return out_hbm.at[row0 + k // 2, pl.ds(_c_off(b), _c_size(b))]

    def fire_gather(k, b):
        pltpu.async_copy(table_hbm.at[idx_ref(k, b)], rows[b], sem_g[b])

    def wait_gather(k, b):
        pltpu.make_async_copy(table_hbm.at[idx_ref(k, b)], rows[b],
                              sem_g[b]).wait()

    def fire_out(k, b):
        pltpu.async_copy(rows[b], out_ref(k, b), sem_o[b])

    def wait_out(k, b):
        pltpu.make_async_copy(rows[b], out_ref(k, b), sem_o[b]).wait()

    for b in range(LA):
        fire_gather(b, b)

    for b in range(NB):
        k = b
        wait_gather(k, b)
        fire_out(k, b)
        bn = (b + LA) % NB
        if k >= LA:
            wait_out(k - LA, bn)
        fire_gather(k + LA, bn)

    def round_body(r, carry):
        k0 = r * NB
        for b in range(NB):
            k = k0 + b
            wait_gather(k, b)
            fire_out(k, b)
            bn = (b + LA) % NB
            wait_out(k - LA, bn)
            fire_gather(k + LA, bn)
        return carry

    lax.fori_loop(1, CPW // NB - 1, round_body, 0)

    k0 = CPW - NB
    for b in range(NB):
        k = k0 + b
        wait_gather(k, b)
        fire_out(k, b)
        if b < LA:
            bn = (b + LA) % NB
            wait_out(k - LA, bn)
            fire_gather(k + LA, bn)

    for b in range(NB):
        wait_out(k0 + b, b)


def kernel(input_, weight):
    return _gather_kernel(input_.astype(jnp.int32), weight)
